# Initial kernel scaffold; baseline (speedup 1.0000x reference)
import jax
import jax.numpy as jnp
from jax.experimental import pallas as pl


def kernel(input_ids, token_type_ids, position_ids, tok_emb, pos_emb, seg_emb, ln_scale, ln_bias):
    raise NotImplementedError("write your pallas kernel here")


# speedup vs baseline: 5.5238x; 5.5238x over previous
import functools

import jax
import jax.numpy as jnp
from jax import lax
from jax.experimental import pallas as pl
from jax.experimental.pallas import tpu as pltpu
from jax.experimental.pallas import tpu_sc as plsc

EPS = 1e-12


GATHER_WINDOW = 128


def _sc_gather_body(table_hbm, i_hbm, o_hbm):
    def body(i_vmem, o_vmem):
        pltpu.sync_copy(table_hbm.at[i_vmem.at[0]], o_vmem)

    num_windows = i_hbm.shape[0]
    pltpu.emit_pipeline(
        body,
        grid=(num_windows,),
        in_specs=[pl.BlockSpec((1, GATHER_WINDOW), index_map=lambda w: (w, 0))],
        out_specs=[pl.BlockSpec((GATHER_WINDOW, table_hbm.shape[1]),
                                index_map=lambda w: (w, 0))],
        core_axis_name=("c", "s"),
        dimension_semantics=(pltpu.PARALLEL,),
    )(i_hbm, o_hbm)


def _sc_gather(table, idx):
    n = idx.shape[0]
    idx2 = idx.reshape(n // GATHER_WINDOW, GATHER_WINDOW)
    mesh = plsc.VectorSubcoreMesh(core_axis_name="c", subcore_axis_name="s")
    kern = pl.kernel(
        _sc_gather_body,
        out_type=jax.ShapeDtypeStruct((n, table.shape[1]), table.dtype),
        mesh=mesh,
    )
    return kern(table, idx2)


ROW_BLOCK = 1024


def _tc_ln_body(g_ref, pid_ref, sid_ref, pe_ref, se_ref, sc_ref, bi_ref, o_ref):
    x = g_ref[...]
    pid = pid_ref[...]
    max_pos = pe_ref.shape[0]
    iota = lax.broadcasted_iota(jnp.int32, (x.shape[0], max_pos), 1)
    onehot = (iota == pid).astype(jnp.bfloat16)
    posv = lax.dot_general(
        onehot, pe_ref[...].astype(jnp.bfloat16),
        dimension_numbers=(((1,), (0,)), ((), ())),
        preferred_element_type=jnp.float32,
    )
    t = sid_ref[...].astype(jnp.float32)
    seg0 = se_ref[0:1, :]
    seg1 = se_ref[1:2, :]
    segv = seg0 + t * (seg1 - seg0)
    x = x + posv + segv
    m = jnp.mean(x, axis=-1, keepdims=True)
    d = x - m
    v = jnp.mean(d * d, axis=-1, keepdims=True)
    normed = d * lax.rsqrt(v + EPS)
    o_ref[...] = normed * sc_ref[...] + bi_ref[...]


def _tc_ln(gathered, pos_ids, seg_ids, pos_emb, seg_emb, ln_scale, ln_bias):
    n, h = gathered.shape
    grid = (n // ROW_BLOCK,)
    return pl.pallas_call(
        _tc_ln_body,
        grid=grid,
        in_specs=[
            pl.BlockSpec((ROW_BLOCK, h), lambda i: (i, 0)),
            pl.BlockSpec((ROW_BLOCK, 1), lambda i: (i, 0)),
            pl.BlockSpec((ROW_BLOCK, 1), lambda i: (i, 0)),
            pl.BlockSpec(pos_emb.shape, lambda i: (0, 0)),
            pl.BlockSpec(seg_emb.shape, lambda i: (0, 0)),
            pl.BlockSpec((1, h), lambda i: (0, 0)),
            pl.BlockSpec((1, h), lambda i: (0, 0)),
        ],
        out_specs=pl.BlockSpec((ROW_BLOCK, h), lambda i: (i, 0)),
        out_shape=jax.ShapeDtypeStruct((n, h), jnp.float32),
        compiler_params=pltpu.CompilerParams(
            dimension_semantics=("parallel",),
        ),
    )(gathered, pos_ids, seg_ids, pos_emb, seg_emb, ln_scale, ln_bias)


def kernel(input_ids, token_type_ids, position_ids, tok_emb, pos_emb, seg_emb,
           ln_scale, ln_bias):
    b, l = input_ids.shape
    h = tok_emb.shape[1]
    n = b * l
    ids = input_ids.reshape(-1).astype(jnp.int32)
    pids = position_ids.reshape(-1, 1).astype(jnp.int32)
    sids = token_type_ids.reshape(-1, 1).astype(jnp.int32)

    gathered = _sc_gather(tok_emb, ids)
    out = _tc_ln(gathered, pids, sids, pos_emb, seg_emb,
                 ln_scale.reshape(1, h), ln_bias.reshape(1, h))
    return out.reshape(b, l, h)


# speedup vs baseline: 6.9179x; 1.2524x over previous
import functools

import jax
import jax.numpy as jnp
from jax import lax
from jax.experimental import pallas as pl
from jax.experimental.pallas import tpu as pltpu
from jax.experimental.pallas import tpu_sc as plsc

EPS = 1e-12


GATHER_WINDOW = 128


def _sc_gather_body(table_hbm, i_hbm, o_hbm):
    def body(i_vmem, o_vmem):
        pltpu.sync_copy(table_hbm.at[i_vmem.at[0]], o_vmem)

    num_windows = i_hbm.shape[0]
    pltpu.emit_pipeline(
        body,
        grid=(num_windows,),
        in_specs=[pl.BlockSpec((1, GATHER_WINDOW), index_map=lambda w: (w, 0))],
        out_specs=[pl.BlockSpec((GATHER_WINDOW, table_hbm.shape[1]),
                                index_map=lambda w: (w, 0))],
        core_axis_name=("c", "s"),
        dimension_semantics=(pltpu.PARALLEL,),
    )(i_hbm, o_hbm)


def _sc_gather(table, idx):
    n = idx.shape[0]
    idx2 = idx.reshape(n // GATHER_WINDOW, GATHER_WINDOW)
    mesh = plsc.VectorSubcoreMesh(core_axis_name="c", subcore_axis_name="s")
    kern = pl.kernel(
        _sc_gather_body,
        out_type=jax.ShapeDtypeStruct((n, table.shape[1]), table.dtype),
        mesh=mesh,
    )
    return kern(table, idx2)


ROW_BLOCK = 1024


def _tc_ln_body(g_ref, pid_ref, sid_ref, pe_ref, se_ref, sc_ref, bi_ref, o_ref):
    x = g_ref[...]
    r = x.shape[0]
    pid = pid_ref[0]
    sid = sid_ref[0]
    max_pos = pe_ref.shape[0]
    iota_p = lax.broadcasted_iota(jnp.int32, (max_pos, r), 0)
    oh_p = (iota_p == pid).astype(jnp.bfloat16)
    posv = lax.dot_general(
        oh_p, pe_ref[...].astype(jnp.bfloat16),
        dimension_numbers=(((0,), (0,)), ((), ())),
        preferred_element_type=jnp.float32,
    )
    iota_s = lax.broadcasted_iota(jnp.int32, (se_ref.shape[0], r), 0)
    oh_s = (iota_s == sid).astype(jnp.bfloat16)
    segv = lax.dot_general(
        oh_s, se_ref[...].astype(jnp.bfloat16),
        dimension_numbers=(((0,), (0,)), ((), ())),
        preferred_element_type=jnp.float32,
    )
    x = x + posv + segv
    m = jnp.mean(x, axis=-1, keepdims=True)
    d = x - m
    v = jnp.mean(d * d, axis=-1, keepdims=True)
    normed = d * lax.rsqrt(v + EPS)
    o_ref[...] = normed * sc_ref[...] + bi_ref[...]


def _tc_ln(gathered, pos_ids, seg_ids, pos_emb, seg_emb, ln_scale, ln_bias):
    n, h = gathered.shape
    grid = (n // ROW_BLOCK,)
    return pl.pallas_call(
        _tc_ln_body,
        grid=grid,
        in_specs=[
            pl.BlockSpec((ROW_BLOCK, h), lambda i: (i, 0)),
            pl.BlockSpec((1, 1, ROW_BLOCK), lambda i: (i, 0, 0)),
            pl.BlockSpec((1, 1, ROW_BLOCK), lambda i: (i, 0, 0)),
            pl.BlockSpec(pos_emb.shape, lambda i: (0, 0)),
            pl.BlockSpec(seg_emb.shape, lambda i: (0, 0)),
            pl.BlockSpec((1, h), lambda i: (0, 0)),
            pl.BlockSpec((1, h), lambda i: (0, 0)),
        ],
        out_specs=pl.BlockSpec((ROW_BLOCK, h), lambda i: (i, 0)),
        out_shape=jax.ShapeDtypeStruct((n, h), jnp.float32),
        compiler_params=pltpu.CompilerParams(
            dimension_semantics=("parallel",),
        ),
    )(gathered, pos_ids, seg_ids, pos_emb, seg_emb, ln_scale, ln_bias)


def kernel(input_ids, token_type_ids, position_ids, tok_emb, pos_emb, seg_emb,
           ln_scale, ln_bias):
    b, l = input_ids.shape
    h = tok_emb.shape[1]
    n = b * l
    ids = input_ids.reshape(-1).astype(jnp.int32)
    pids = position_ids.reshape(n // ROW_BLOCK, 1, ROW_BLOCK).astype(jnp.int32)
    sids = token_type_ids.reshape(n // ROW_BLOCK, 1, ROW_BLOCK).astype(jnp.int32)

    gathered = _sc_gather(tok_emb, ids)
    out = _tc_ln(gathered, pids, sids, pos_emb, seg_emb,
                 ln_scale.reshape(1, h), ln_bias.reshape(1, h))
    return out.reshape(b, l, h)
